# Initial kernel scaffold; baseline (speedup 1.0000x reference)
import jax
import jax.numpy as jnp
from jax.experimental import pallas as pl


def kernel(x_enc, x_mark_enc, x_dec, x_mark_dec, W_patch, Wg, W1, b1, W2, b2, Wh, bh):
    raise NotImplementedError("write your pallas kernel here")


# speedup vs baseline: 2.6123x; 2.6123x over previous
import functools

import numpy as np
import jax
import jax.numpy as jnp
from jax import lax
from jax.experimental import pallas as pl
from jax.experimental.pallas import tpu as pltpu
from jax.experimental.pallas import tpu_sc as plsc

_B, _L, _NV, _DM = 8, 512, 7, 768
_PLEN, _ST, _E = 16, 8, 8
_HID, _PRED, _NP = 1536, 96, 64
_BV = _B * _NV
_T = _BV * _NP
_BLK = 512
_NBLK = _T // _BLK
_TILE = 128
_GCAP = 8192
_NTILE = _GCAP // _TILE
_NW = 32
_CHUNK = _T // _NW


def _posemb_np():
    position = np.arange(_NP, dtype=np.float32)[:, None]
    div = np.exp(np.arange(0, _DM, 2, dtype=np.float32) * -(np.log(10000.0) / _DM))
    pe = np.zeros((_NP, _DM), np.float32)
    pe[:, 0::2] = np.sin(position * div)
    pe[:, 1::2] = np.cos(position * div)
    return pe


_POS = _posemb_np()


def _prologue_body(patches_ref, wp_ref, pos_ref, wg_ref, tril_ref,
                   emb_ref, i0_ref, i1_ref, v0_ref, v1_ref, r0_ref, r1_ref,
                   cnt_ref, sp_ref, sl_ref, acc):
    i = pl.program_id(0)

    @pl.when(i == 0)
    def _():
        acc[...] = jnp.zeros((8, 128), jnp.float32)

    p = patches_ref[...]
    emb = jnp.dot(p, wp_ref[...], preferred_element_type=jnp.float32) + pos_ref[...]
    emb_ref[...] = emb
    logits = jnp.dot(emb, wg_ref[...], preferred_element_type=jnp.float32)
    m = jnp.max(logits, axis=1, keepdims=True)
    ex = jnp.exp(logits - m)
    s = jnp.sum(ex, axis=1, keepdims=True)
    probs = ex / s
    lse = jnp.log(s) + m
    iota8 = lax.broadcasted_iota(jnp.int32, (_BLK, _E), 1)
    v0 = jnp.max(probs, axis=1, keepdims=True)
    i0 = jnp.min(jnp.where(probs == v0, iota8, _E), axis=1, keepdims=True)
    m0 = iota8 == i0
    pm = jnp.where(m0, -jnp.inf, probs)
    v1 = jnp.max(pm, axis=1, keepdims=True)
    i1 = jnp.min(jnp.where(pm == v1, iota8, _E), axis=1, keepdims=True)
    m1 = iota8 == i1
    d = m0.astype(jnp.float32) + m1.astype(jnp.float32)
    incl = jnp.dot(tril_ref[...], d, precision=lax.Precision.HIGHEST,
                   preferred_element_type=jnp.float32)
    cnt = acc[0:1, 0:_E]
    excl = incl - d + cnt
    r0 = jnp.sum(excl * m0.astype(jnp.float32), axis=1, keepdims=True)
    r1 = jnp.sum(excl * m1.astype(jnp.float32), axis=1, keepdims=True)
    i0_ref[...] = i0
    i1_ref[...] = i1
    v0_ref[...] = v0
    v1_ref[...] = v1
    r0_ref[...] = r0.astype(jnp.int32)
    r1_ref[...] = r1.astype(jnp.int32)
    ncnt = cnt + jnp.sum(d, axis=0, keepdims=True)
    nsp = acc[1:2, 0:_E] + jnp.sum(probs, axis=0, keepdims=True)
    nsl = acc[2:3, 0:1] + jnp.sum(lse * lse)
    acc[0:1, 0:_E] = ncnt
    acc[1:2, 0:_E] = nsp
    acc[2:3, 0:1] = nsl
    cnt_ref[...] = ncnt
    sp_ref[...] = nsp
    sl_ref[...] = nsl


def _prologue(patches, wp, posrep, wg, tril):
    return pl.pallas_call(
        _prologue_body,
        grid=(_NBLK,),
        in_specs=[
            pl.BlockSpec((_BLK, _PLEN), lambda i: (i, 0)),
            pl.BlockSpec((_PLEN, _DM), lambda i: (0, 0)),
            pl.BlockSpec((_BLK, _DM), lambda i: (0, 0)),
            pl.BlockSpec((_DM, _E), lambda i: (0, 0)),
            pl.BlockSpec((_BLK, _BLK), lambda i: (0, 0)),
        ],
        out_specs=[
            pl.BlockSpec((_BLK, _DM), lambda i: (i, 0)),
            pl.BlockSpec((_BLK, 1), lambda i: (i, 0)),
            pl.BlockSpec((_BLK, 1), lambda i: (i, 0)),
            pl.BlockSpec((_BLK, 1), lambda i: (i, 0)),
            pl.BlockSpec((_BLK, 1), lambda i: (i, 0)),
            pl.BlockSpec((_BLK, 1), lambda i: (i, 0)),
            pl.BlockSpec((_BLK, 1), lambda i: (i, 0)),
            pl.BlockSpec((1, _E), lambda i: (0, 0)),
            pl.BlockSpec((1, _E), lambda i: (0, 0)),
            pl.BlockSpec((1, 1), lambda i: (0, 0)),
        ],
        out_shape=[
            jax.ShapeDtypeStruct((_T, _DM), jnp.float32),
            jax.ShapeDtypeStruct((_T, 1), jnp.int32),
            jax.ShapeDtypeStruct((_T, 1), jnp.int32),
            jax.ShapeDtypeStruct((_T, 1), jnp.float32),
            jax.ShapeDtypeStruct((_T, 1), jnp.float32),
            jax.ShapeDtypeStruct((_T, 1), jnp.int32),
            jax.ShapeDtypeStruct((_T, 1), jnp.int32),
            jax.ShapeDtypeStruct((1, _E), jnp.float32),
            jax.ShapeDtypeStruct((1, _E), jnp.float32),
            jax.ShapeDtypeStruct((1, 1), jnp.float32),
        ],
        scratch_shapes=[pltpu.VMEM((8, 128), jnp.float32)],
    )(patches, wp, posrep, wg, tril)


def _sc_dispatch(emb, ga, gb, dest_a, dest_b):
    mesh = plsc.VectorSubcoreMesh(core_axis_name="c", subcore_axis_name="s")

    @functools.partial(
        pl.kernel,
        out_type=[jax.ShapeDtypeStruct((_GCAP, _DM), jnp.float32),
                  jax.ShapeDtypeStruct((_GCAP, 128), jnp.float32)],
        mesh=mesh,
        scratch_types=[
            pltpu.VMEM((_CHUNK, _DM), jnp.float32),
            pltpu.VMEM((_CHUNK, 128), jnp.float32),
            pltpu.VMEM((_CHUNK, 128), jnp.float32),
            pltpu.VMEM((_CHUNK,), jnp.int32),
            pltpu.VMEM((_CHUNK,), jnp.int32),
            pltpu.SemaphoreType.DMA,
        ],
    )
    def k(emb_hbm, ga_hbm, gb_hbm, da_hbm, db_hbm, xg_hbm, xgate_hbm,
          emb_v, ga_v, gb_v, ia_v, ib_v, sem):
        wid = lax.axis_index("s") * 2 + lax.axis_index("c")
        base = wid * _CHUNK
        pltpu.sync_copy(emb_hbm.at[pl.ds(base, _CHUNK)], emb_v)
        pltpu.sync_copy(ga_hbm.at[pl.ds(base, _CHUNK)], ga_v)
        pltpu.sync_copy(gb_hbm.at[pl.ds(base, _CHUNK)], gb_v)
        pltpu.sync_copy(da_hbm.at[wid], ia_v)
        pltpu.sync_copy(db_hbm.at[wid], ib_v)
        c1 = pltpu.async_copy(emb_v, xg_hbm.at[ia_v], sem)
        c2 = pltpu.async_copy(emb_v, xg_hbm.at[ib_v], sem)
        c3 = pltpu.async_copy(ga_v, xgate_hbm.at[ia_v], sem)
        c4 = pltpu.async_copy(gb_v, xgate_hbm.at[ib_v], sem)
        c1.wait()
        c2.wait()
        c3.wait()
        c4.wait()

    return k(emb, ga, gb, dest_a, dest_b)


def _ffn_body(te_ref, x_ref, g_ref, w1_ref, b1_ref, w2_ref, b2_ref, y_ref):
    x = x_ref[...]
    h = jnp.dot(x, w1_ref[...], preferred_element_type=jnp.float32) + b1_ref[...]
    h = h * 0.5 * (1.0 + lax.erf(h * np.float32(1.0 / np.sqrt(2.0))))
    y = jnp.dot(h, w2_ref[...], preferred_element_type=jnp.float32) + b2_ref[...]
    y_ref[...] = y * g_ref[:, 0:1]


def _ffn(tile_expert, xg, xgate, w1, b1, w2, b2):
    grid_spec = pltpu.PrefetchScalarGridSpec(
        num_scalar_prefetch=1,
        grid=(_NTILE,),
        in_specs=[
            pl.BlockSpec((_TILE, _DM), lambda i, te: (i, 0)),
            pl.BlockSpec((_TILE, 128), lambda i, te: (i, 0)),
            pl.BlockSpec((None, _DM, _HID), lambda i, te: (te[i], 0, 0)),
            pl.BlockSpec((None, 1, _HID), lambda i, te: (te[i], 0, 0)),
            pl.BlockSpec((None, _HID, _DM), lambda i, te: (te[i], 0, 0)),
            pl.BlockSpec((None, 1, _DM), lambda i, te: (te[i], 0, 0)),
        ],
        out_specs=pl.BlockSpec((_TILE, _DM), lambda i, te: (i, 0)),
    )
    return pl.pallas_call(
        _ffn_body,
        grid_spec=grid_spec,
        out_shape=jax.ShapeDtypeStruct((_GCAP, _DM), jnp.float32),
    )(tile_expert, xg, xgate, w1,
      b1.reshape(_E, 1, _HID), w2, b2.reshape(_E, 1, _DM))


def _sc_combine(y, da_p, db_p):
    mesh = plsc.VectorSubcoreMesh(core_axis_name="c", subcore_axis_name="s")

    @functools.partial(
        pl.kernel,
        out_type=[jax.ShapeDtypeStruct((_T, _DM), jnp.float32),
                  jax.ShapeDtypeStruct((_T, _DM), jnp.float32)],
        mesh=mesh,
        scratch_types=[
            pltpu.VMEM((_CHUNK, _DM), jnp.float32),
            pltpu.VMEM((_CHUNK,), jnp.int32),
            pltpu.VMEM((_CHUNK,), jnp.int32),
            pltpu.SemaphoreType.DMA,
        ],
    )
    def k(y_hbm, da_hbm, db_hbm, ya_hbm, yb_hbm, buf_v, ia_v, ib_v, sem):
        wid = lax.axis_index("s") * 2 + lax.axis_index("c")
        base = wid * _CHUNK
        pltpu.sync_copy(da_hbm.at[wid], ia_v)
        pltpu.sync_copy(db_hbm.at[wid], ib_v)
        pltpu.async_copy(y_hbm.at[ia_v], buf_v, sem).wait()
        pltpu.sync_copy(buf_v, ya_hbm.at[pl.ds(base, _CHUNK)])
        pltpu.async_copy(y_hbm.at[ib_v], buf_v, sem).wait()
        pltpu.sync_copy(buf_v, yb_hbm.at[pl.ds(base, _CHUNK)])

    return k(y, da_p, db_p)


def _head_body(ya_ref, yb_ref, wh_ref, bh_ref, out_ref):
    i = pl.program_id(0)

    @pl.when(i == 0)
    def _():
        out_ref[...] = jnp.broadcast_to(bh_ref[...], (_BV, _PRED))

    s = ya_ref[...] + yb_ref[...]
    out_ref[...] += jnp.dot(s, wh_ref[...], preferred_element_type=jnp.float32)


def _head(yan, ybn, wh2n, bh2):
    return pl.pallas_call(
        _head_body,
        grid=(_NP,),
        in_specs=[
            pl.BlockSpec((_BV, _DM), lambda i: (i, 0)),
            pl.BlockSpec((_BV, _DM), lambda i: (i, 0)),
            pl.BlockSpec((None, _DM, _PRED), lambda i: (i, 0, 0)),
            pl.BlockSpec((1, _PRED), lambda i: (0, 0)),
        ],
        out_specs=pl.BlockSpec((_BV, _PRED), lambda i: (0, 0)),
        out_shape=jax.ShapeDtypeStruct((_BV, _PRED), jnp.float32),
    )(yan, ybn, wh2n, bh2)


def kernel(x_enc, x_mark_enc, x_dec, x_mark_dec, W_patch, Wg, W1, b1, W2, b2, Wh, bh):
    means = jnp.mean(x_enc, axis=1, keepdims=True)
    xe = x_enc - means
    stdev = jnp.sqrt(jnp.var(xe, axis=1, keepdims=True) + 1e-5)
    xe = xe / stdev
    x = jnp.transpose(xe, (0, 2, 1))
    xpad = jnp.concatenate([x, jnp.repeat(x[..., -1:], _ST, axis=-1)], axis=-1)
    xc = xpad.reshape(_BV, _L // _ST + 1, _ST)
    patches = jnp.concatenate([xc[:, :-1, :], xc[:, 1:, :]], axis=-1)
    patches = patches.reshape(_T, _PLEN)

    posrep = jnp.asarray(np.tile(_POS, (_BLK // _NP, 1)))
    tril = jnp.asarray(np.tril(np.ones((_BLK, _BLK), np.float32)))
    (emb, i0, i1, v0, v1, r0, r1, cnt, sp, sl) = _prologue(
        patches, W_patch, posrep, Wg, tril)

    counts = cnt[0].astype(jnp.int32)
    cap = ((counts + _TILE - 1) // _TILE) * _TILE
    po = jnp.concatenate([jnp.zeros((1,), jnp.int32), jnp.cumsum(cap)])
    dest0 = po[i0[:, 0]] + r0[:, 0]
    dest1 = po[i1[:, 0]] + r1[:, 0]
    tile_start = jnp.arange(_NTILE, dtype=jnp.int32) * _TILE
    tile_expert = jnp.clip(
        jnp.searchsorted(po[1:], tile_start, side="right"), 0, _E - 1
    ).astype(jnp.int32)

    ga = jnp.broadcast_to(v0, (_T, 128))
    gb = jnp.broadcast_to(v1, (_T, 128))
    xg, xgate = _sc_dispatch(emb, ga, gb,
                             dest0.reshape(_NW, _CHUNK),
                             dest1.reshape(_NW, _CHUNK))

    y = _ffn(tile_expert, xg, xgate, W1, b1, W2, b2)

    tok = jnp.asarray((np.arange(_T) % _BV) * _NP + np.arange(_T) // _BV)
    yan, ybn = _sc_combine(y,
                           dest0[tok].reshape(_NW, _CHUNK),
                           dest1[tok].reshape(_NW, _CHUNK))

    wh2n = Wh.reshape(_DM, _NP, _PRED).transpose(1, 0, 2)
    dec56 = _head(yan, ybn, wh2n, bh.reshape(1, _PRED))
    dec = dec56.reshape(_B, _NV, _PRED).transpose(0, 2, 1)
    dec = dec * stdev[:, 0, :][:, None, :] + means[:, 0, :][:, None, :]

    f_mean = cnt[0] / _T
    p_mean = sp[0] / _T
    balance = _E * jnp.sum(f_mean * p_mean)
    zloss = sl[0, 0] / _T
    total_aux = 0.01 * balance + 0.001 * zloss
    return dec, total_aux


# speedup vs baseline: 2.8366x; 1.0859x over previous
import functools

import numpy as np
import jax
import jax.numpy as jnp
from jax import lax
from jax.experimental import pallas as pl
from jax.experimental.pallas import tpu as pltpu
from jax.experimental.pallas import tpu_sc as plsc

_B, _L, _NV, _DM = 8, 512, 7, 768
_PLEN, _ST, _E = 16, 8, 8
_HID, _PRED, _NP = 1536, 96, 64
_BV = _B * _NV
_T = _BV * _NP
_BLK = 512
_NBLK = _T // _BLK
_TILE = 128
_GCAP = 8192
_NTILE = _GCAP // _TILE
_NW = 32
_CHUNK = _T // _NW
_HG = 8


def _posemb_np():
    position = np.arange(_NP, dtype=np.float32)[:, None]
    div = np.exp(np.arange(0, _DM, 2, dtype=np.float32) * -(np.log(10000.0) / _DM))
    pe = np.zeros((_NP, _DM), np.float32)
    pe[:, 0::2] = np.sin(position * div)
    pe[:, 1::2] = np.cos(position * div)
    return pe


_POS_NMAJOR = np.repeat(_posemb_np(), _BV, axis=0)


def _prologue_body(patches_ref, wp_ref, pos_ref, wg_ref, tril_ref,
                   emb_ref, i0_ref, i1_ref, v0_ref, v1_ref, r0_ref, r1_ref,
                   cnt_ref, sp_ref, sl_ref, acc):
    i = pl.program_id(0)

    @pl.when(i == 0)
    def _():
        acc[...] = jnp.zeros((8, 128), jnp.float32)

    p = patches_ref[...]
    emb = jnp.dot(p, wp_ref[...], preferred_element_type=jnp.float32) + pos_ref[...]
    emb_ref[...] = emb
    logits = jnp.dot(emb, wg_ref[...], preferred_element_type=jnp.float32)
    m = jnp.max(logits, axis=1, keepdims=True)
    ex = jnp.exp(logits - m)
    s = jnp.sum(ex, axis=1, keepdims=True)
    probs = ex / s
    lse = jnp.log(s) + m
    iota8 = lax.broadcasted_iota(jnp.int32, (_BLK, _E), 1)
    v0 = jnp.max(probs, axis=1, keepdims=True)
    i0 = jnp.min(jnp.where(probs == v0, iota8, _E), axis=1, keepdims=True)
    m0 = iota8 == i0
    pm = jnp.where(m0, -jnp.inf, probs)
    v1 = jnp.max(pm, axis=1, keepdims=True)
    i1 = jnp.min(jnp.where(pm == v1, iota8, _E), axis=1, keepdims=True)
    m1 = iota8 == i1
    d = m0.astype(jnp.float32) + m1.astype(jnp.float32)
    incl = jnp.dot(tril_ref[...], d, precision=lax.Precision.HIGHEST,
                   preferred_element_type=jnp.float32)
    cnt = acc[0:1, 0:_E]
    excl = incl - d + cnt
    r0 = jnp.sum(excl * m0.astype(jnp.float32), axis=1, keepdims=True)
    r1 = jnp.sum(excl * m1.astype(jnp.float32), axis=1, keepdims=True)
    i0_ref[...] = i0
    i1_ref[...] = i1
    v0_ref[...] = v0
    v1_ref[...] = v1
    r0_ref[...] = r0.astype(jnp.int32)
    r1_ref[...] = r1.astype(jnp.int32)
    ncnt = cnt + jnp.sum(d, axis=0, keepdims=True)
    nsp = acc[1:2, 0:_E] + jnp.sum(probs, axis=0, keepdims=True)
    nsl = acc[2:3, 0:1] + jnp.sum(lse * lse)
    acc[0:1, 0:_E] = ncnt
    acc[1:2, 0:_E] = nsp
    acc[2:3, 0:1] = nsl
    cnt_ref[...] = ncnt
    sp_ref[...] = nsp
    sl_ref[...] = nsl


def _prologue(patches, wp, posrep, wg, tril):
    return pl.pallas_call(
        _prologue_body,
        grid=(_NBLK,),
        in_specs=[
            pl.BlockSpec((_BLK, _PLEN), lambda i: (i, 0)),
            pl.BlockSpec((_PLEN, _DM), lambda i: (0, 0)),
            pl.BlockSpec((_BLK, _DM), lambda i: (i, 0)),
            pl.BlockSpec((_DM, _E), lambda i: (0, 0)),
            pl.BlockSpec((_BLK, _BLK), lambda i: (0, 0)),
        ],
        out_specs=[
            pl.BlockSpec((_BLK, _DM), lambda i: (i, 0)),
            pl.BlockSpec((_BLK, 1), lambda i: (i, 0)),
            pl.BlockSpec((_BLK, 1), lambda i: (i, 0)),
            pl.BlockSpec((_BLK, 1), lambda i: (i, 0)),
            pl.BlockSpec((_BLK, 1), lambda i: (i, 0)),
            pl.BlockSpec((_BLK, 1), lambda i: (i, 0)),
            pl.BlockSpec((_BLK, 1), lambda i: (i, 0)),
            pl.BlockSpec((1, _E), lambda i: (0, 0)),
            pl.BlockSpec((1, _E), lambda i: (0, 0)),
            pl.BlockSpec((1, 1), lambda i: (0, 0)),
        ],
        out_shape=[
            jax.ShapeDtypeStruct((_T, _DM), jnp.float32),
            jax.ShapeDtypeStruct((_T, 1), jnp.int32),
            jax.ShapeDtypeStruct((_T, 1), jnp.int32),
            jax.ShapeDtypeStruct((_T, 1), jnp.float32),
            jax.ShapeDtypeStruct((_T, 1), jnp.float32),
            jax.ShapeDtypeStruct((_T, 1), jnp.int32),
            jax.ShapeDtypeStruct((_T, 1), jnp.int32),
            jax.ShapeDtypeStruct((1, _E), jnp.float32),
            jax.ShapeDtypeStruct((1, _E), jnp.float32),
            jax.ShapeDtypeStruct((1, 1), jnp.float32),
        ],
        scratch_shapes=[pltpu.VMEM((8, 128), jnp.float32)],
    )(patches, wp, posrep, wg, tril)


def _finish_body(cnt_ref, sp_ref, sl_ref, i0_ref, i1_ref, r0_ref, r1_ref,
                 up_ref, d0_ref, d1_ref, te_ref, aux_ref):
    cnt = cnt_ref[...]
    cap = jnp.floor((cnt + 127.0) * (1.0 / 128.0)) * 128.0
    po = jnp.dot(cap, up_ref[...], precision=lax.Precision.HIGHEST,
                 preferred_element_type=jnp.float32)
    iota8 = lax.broadcasted_iota(jnp.int32, (_T, _E), 1)
    one0 = (iota8 == i0_ref[...]).astype(jnp.float32)
    one1 = (iota8 == i1_ref[...]).astype(jnp.float32)
    d0 = jnp.sum(one0 * po, axis=1, keepdims=True).astype(jnp.int32) + r0_ref[...]
    d1 = jnp.sum(one1 * po, axis=1, keepdims=True).astype(jnp.int32) + r1_ref[...]
    d0_ref[...] = d0
    d1_ref[...] = d1
    ts = lax.broadcasted_iota(jnp.int32, (_NTILE, _E), 0).astype(jnp.float32) * float(_TILE)
    ge = (ts >= po).astype(jnp.int32)
    te_ref[...] = jnp.sum(ge, axis=1, keepdims=True) - 1
    balance = float(_E) * jnp.sum(cnt * sp_ref[...]) * (1.0 / (_T * _T))
    aux_ref[...] = 0.01 * balance + 0.001 * (sl_ref[...] * (1.0 / _T))


def _finish(cnt, sp, sl, i0, i1, r0, r1):
    up = jnp.asarray(np.triu(np.ones((_E, _E), np.float32), 1))
    return pl.pallas_call(
        _finish_body,
        out_shape=[
            jax.ShapeDtypeStruct((_T, 1), jnp.int32),
            jax.ShapeDtypeStruct((_T, 1), jnp.int32),
            jax.ShapeDtypeStruct((_NTILE, 1), jnp.int32),
            jax.ShapeDtypeStruct((1, 1), jnp.float32),
        ],
    )(cnt, sp, sl, i0, i1, r0, r1, up)


def _sc_dispatch(emb, ga, gb, dest_a, dest_b):
    mesh = plsc.VectorSubcoreMesh(core_axis_name="c", subcore_axis_name="s")

    @functools.partial(
        pl.kernel,
        out_type=[jax.ShapeDtypeStruct((_GCAP, _DM), jnp.float32),
                  jax.ShapeDtypeStruct((_GCAP, 128), jnp.float32)],
        mesh=mesh,
        scratch_types=[
            pltpu.VMEM((_CHUNK, _DM), jnp.float32),
            pltpu.VMEM((_CHUNK, 128), jnp.float32),
            pltpu.VMEM((_CHUNK, 128), jnp.float32),
            pltpu.VMEM((_CHUNK,), jnp.int32),
            pltpu.VMEM((_CHUNK,), jnp.int32),
            pltpu.SemaphoreType.DMA,
        ],
    )
    def k(emb_hbm, ga_hbm, gb_hbm, da_hbm, db_hbm, xg_hbm, xgate_hbm,
          emb_v, ga_v, gb_v, ia_v, ib_v, sem):
        wid = lax.axis_index("s") * 2 + lax.axis_index("c")
        base = wid * _CHUNK
        pltpu.sync_copy(emb_hbm.at[pl.ds(base, _CHUNK)], emb_v)
        pltpu.sync_copy(ga_hbm.at[pl.ds(base, _CHUNK)], ga_v)
        pltpu.sync_copy(gb_hbm.at[pl.ds(base, _CHUNK)], gb_v)
        pltpu.sync_copy(da_hbm.at[wid], ia_v)
        pltpu.sync_copy(db_hbm.at[wid], ib_v)
        c1 = pltpu.async_copy(emb_v, xg_hbm.at[ia_v], sem)
        c2 = pltpu.async_copy(emb_v, xg_hbm.at[ib_v], sem)
        c3 = pltpu.async_copy(ga_v, xgate_hbm.at[ia_v], sem)
        c4 = pltpu.async_copy(gb_v, xgate_hbm.at[ib_v], sem)
        c1.wait()
        c2.wait()
        c3.wait()
        c4.wait()

    return k(emb, ga, gb, dest_a, dest_b)


def _ffn_body(te_ref, x_ref, g_ref, w1_ref, b1_ref, w2_ref, b2_ref, y_ref):
    x = x_ref[...].astype(jnp.bfloat16)
    h = jnp.dot(x, w1_ref[...], preferred_element_type=jnp.float32) + b1_ref[...]
    h = h * 0.5 * (1.0 + lax.erf(h * np.float32(1.0 / np.sqrt(2.0))))
    h = h.astype(jnp.bfloat16)
    y = jnp.dot(h, w2_ref[...], preferred_element_type=jnp.float32) + b2_ref[...]
    y_ref[...] = y * g_ref[:, 0:1]


def _ffn(tile_expert, xg, xgate, w1, b1, w2, b2):
    grid_spec = pltpu.PrefetchScalarGridSpec(
        num_scalar_prefetch=1,
        grid=(_NTILE,),
        in_specs=[
            pl.BlockSpec((_TILE, _DM), lambda i, te: (i, 0)),
            pl.BlockSpec((_TILE, 128), lambda i, te: (i, 0)),
            pl.BlockSpec((None, _DM, _HID), lambda i, te: (te[i], 0, 0)),
            pl.BlockSpec((None, 1, _HID), lambda i, te: (te[i], 0, 0)),
            pl.BlockSpec((None, _HID, _DM), lambda i, te: (te[i], 0, 0)),
            pl.BlockSpec((None, 1, _DM), lambda i, te: (te[i], 0, 0)),
        ],
        out_specs=pl.BlockSpec((_TILE, _DM), lambda i, te: (i, 0)),
    )
    return pl.pallas_call(
        _ffn_body,
        grid_spec=grid_spec,
        out_shape=jax.ShapeDtypeStruct((_GCAP, _DM), jnp.float32),
    )(tile_expert, xg, xgate,
      w1.astype(jnp.bfloat16), b1.reshape(_E, 1, _HID),
      w2.astype(jnp.bfloat16), b2.reshape(_E, 1, _DM))


def _sc_combine(y, da_p, db_p):
    mesh = plsc.VectorSubcoreMesh(core_axis_name="c", subcore_axis_name="s")

    @functools.partial(
        pl.kernel,
        out_type=[jax.ShapeDtypeStruct((_T, _DM), jnp.float32),
                  jax.ShapeDtypeStruct((_T, _DM), jnp.float32)],
        mesh=mesh,
        scratch_types=[
            pltpu.VMEM((_CHUNK, _DM), jnp.float32),
            pltpu.VMEM((_CHUNK,), jnp.int32),
            pltpu.VMEM((_CHUNK,), jnp.int32),
            pltpu.SemaphoreType.DMA,
        ],
    )
    def k(y_hbm, da_hbm, db_hbm, ya_hbm, yb_hbm, buf_v, ia_v, ib_v, sem):
        wid = lax.axis_index("s") * 2 + lax.axis_index("c")
        base = wid * _CHUNK
        pltpu.sync_copy(da_hbm.at[wid], ia_v)
        pltpu.sync_copy(db_hbm.at[wid], ib_v)
        pltpu.async_copy(y_hbm.at[ia_v], buf_v, sem).wait()
        pltpu.sync_copy(buf_v, ya_hbm.at[pl.ds(base, _CHUNK)])
        pltpu.async_copy(y_hbm.at[ib_v], buf_v, sem).wait()
        pltpu.sync_copy(buf_v, yb_hbm.at[pl.ds(base, _CHUNK)])

    return k(y, da_p, db_p)


def _head_body(ya_ref, yb_ref, wh_ref, bh_ref, out_ref):
    i = pl.program_id(0)

    @pl.when(i == 0)
    def _():
        out_ref[...] = jnp.broadcast_to(bh_ref[...], (_BV, _PRED))

    s = ya_ref[...] + yb_ref[...]
    acc = out_ref[...]
    for k in range(_HG):
        acc = acc + jnp.dot(s[k * _BV:(k + 1) * _BV, :], wh_ref[k],
                            preferred_element_type=jnp.float32)
    out_ref[...] = acc


def _head(yan, ybn, wh2n, bh2):
    return pl.pallas_call(
        _head_body,
        grid=(_NP // _HG,),
        in_specs=[
            pl.BlockSpec((_HG * _BV, _DM), lambda i: (i, 0)),
            pl.BlockSpec((_HG * _BV, _DM), lambda i: (i, 0)),
            pl.BlockSpec((_HG, _DM, _PRED), lambda i: (i, 0, 0)),
            pl.BlockSpec((1, _PRED), lambda i: (0, 0)),
        ],
        out_specs=pl.BlockSpec((_BV, _PRED), lambda i: (0, 0)),
        out_shape=jax.ShapeDtypeStruct((_BV, _PRED), jnp.float32),
    )(yan, ybn, wh2n, bh2)


def kernel(x_enc, x_mark_enc, x_dec, x_mark_dec, W_patch, Wg, W1, b1, W2, b2, Wh, bh):
    means = jnp.mean(x_enc, axis=1, keepdims=True)
    xe = x_enc - means
    stdev = jnp.sqrt(jnp.var(xe, axis=1, keepdims=True) + 1e-5)
    xe = xe / stdev
    x = jnp.transpose(xe, (0, 2, 1))
    xpad = jnp.concatenate([x, jnp.repeat(x[..., -1:], _ST, axis=-1)], axis=-1)
    xc = xpad.reshape(_BV, _L // _ST + 1, _ST)
    patches = jnp.concatenate([xc[:, :-1, :], xc[:, 1:, :]], axis=-1)
    patches = patches.transpose(1, 0, 2).reshape(_T, _PLEN)

    posrep = jnp.asarray(_POS_NMAJOR)
    tril = jnp.asarray(np.tril(np.ones((_BLK, _BLK), np.float32)))
    (emb, i0, i1, v0, v1, r0, r1, cnt, sp, sl) = _prologue(
        patches, W_patch, posrep, Wg, tril)

    dest0, dest1, te, aux = _finish(cnt, sp, sl, i0, i1, r0, r1)
    tile_expert = te.reshape(_NTILE)

    ga = jnp.broadcast_to(v0, (_T, 128))
    gb = jnp.broadcast_to(v1, (_T, 128))
    xg, xgate = _sc_dispatch(emb, ga, gb,
                             dest0.reshape(_NW, _CHUNK),
                             dest1.reshape(_NW, _CHUNK))

    y = _ffn(tile_expert, xg, xgate, W1, b1, W2, b2)

    yan, ybn = _sc_combine(y,
                           dest0.reshape(_NW, _CHUNK),
                           dest1.reshape(_NW, _CHUNK))

    wh2n = Wh.reshape(_DM, _NP, _PRED).transpose(1, 0, 2)
    dec56 = _head(yan, ybn, wh2n, bh.reshape(1, _PRED))
    dec = dec56.reshape(_B, _NV, _PRED).transpose(0, 2, 1)
    dec = dec * stdev[:, 0, :][:, None, :] + means[:, 0, :][:, None, :]

    return dec, aux[0, 0]


# speedup vs baseline: 2.9471x; 1.0389x over previous
import functools

import numpy as np
import jax
import jax.numpy as jnp
from jax import lax
from jax.experimental import pallas as pl
from jax.experimental.pallas import tpu as pltpu
from jax.experimental.pallas import tpu_sc as plsc

_B, _L, _NV, _DM = 8, 512, 7, 768
_PLEN, _ST, _E = 16, 8, 8
_HID, _PRED, _NP = 1536, 96, 64
_BV = _B * _NV
_T = _BV * _NP
_BLK = 512
_NBLK = _T // _BLK
_TILE = 128
_GCAP = 8192
_NTILE = _GCAP // _TILE
_NW = 32
_CHUNK = _T // _NW
_HG = 8


def _posemb_np():
    position = np.arange(_NP, dtype=np.float32)[:, None]
    div = np.exp(np.arange(0, _DM, 2, dtype=np.float32) * -(np.log(10000.0) / _DM))
    pe = np.zeros((_NP, _DM), np.float32)
    pe[:, 0::2] = np.sin(position * div)
    pe[:, 1::2] = np.cos(position * div)
    return pe


_POS_NMAJOR = np.repeat(_posemb_np(), _BV, axis=0)


def _prologue_body(patches_ref, wp_ref, pos_ref, wg_ref, tril_ref,
                   emb_ref, i0_ref, i1_ref, v0_ref, v1_ref, r0_ref, r1_ref,
                   cnt_ref, sp_ref, sl_ref, acc):
    i = pl.program_id(0)

    @pl.when(i == 0)
    def _():
        acc[...] = jnp.zeros((8, 128), jnp.float32)

    p = patches_ref[...]
    emb = jnp.dot(p, wp_ref[...], preferred_element_type=jnp.float32) + pos_ref[...]
    emb_ref[...] = emb
    logits = jnp.dot(emb, wg_ref[...], preferred_element_type=jnp.float32)
    m = jnp.max(logits, axis=1, keepdims=True)
    ex = jnp.exp(logits - m)
    s = jnp.sum(ex, axis=1, keepdims=True)
    probs = ex / s
    lse = jnp.log(s) + m
    iota8 = lax.broadcasted_iota(jnp.int32, (_BLK, _E), 1)
    v0 = jnp.max(probs, axis=1, keepdims=True)
    i0 = jnp.min(jnp.where(probs == v0, iota8, _E), axis=1, keepdims=True)
    m0 = iota8 == i0
    pm = jnp.where(m0, -jnp.inf, probs)
    v1 = jnp.max(pm, axis=1, keepdims=True)
    i1 = jnp.min(jnp.where(pm == v1, iota8, _E), axis=1, keepdims=True)
    m1 = iota8 == i1
    d = m0.astype(jnp.float32) + m1.astype(jnp.float32)
    incl = jnp.dot(tril_ref[...], d, precision=lax.Precision.HIGHEST,
                   preferred_element_type=jnp.float32)
    cnt = acc[0:1, 0:_E]
    excl = incl - d + cnt
    r0 = jnp.sum(excl * m0.astype(jnp.float32), axis=1, keepdims=True)
    r1 = jnp.sum(excl * m1.astype(jnp.float32), axis=1, keepdims=True)
    i0_ref[...] = i0
    i1_ref[...] = i1
    v0_ref[...] = v0
    v1_ref[...] = v1
    r0_ref[...] = r0.astype(jnp.int32)
    r1_ref[...] = r1.astype(jnp.int32)
    ncnt = cnt + jnp.sum(d, axis=0, keepdims=True)
    nsp = acc[1:2, 0:_E] + jnp.sum(probs, axis=0, keepdims=True)
    nsl = acc[2:3, 0:1] + jnp.sum(lse * lse)
    acc[0:1, 0:_E] = ncnt
    acc[1:2, 0:_E] = nsp
    acc[2:3, 0:1] = nsl
    cnt_ref[...] = ncnt
    sp_ref[...] = nsp
    sl_ref[...] = nsl


def _prologue(patches, wp, posrep, wg, tril):
    return pl.pallas_call(
        _prologue_body,
        grid=(_NBLK,),
        in_specs=[
            pl.BlockSpec((_BLK, _PLEN), lambda i: (i, 0)),
            pl.BlockSpec((_PLEN, _DM), lambda i: (0, 0)),
            pl.BlockSpec((_BLK, _DM), lambda i: (i, 0)),
            pl.BlockSpec((_DM, _E), lambda i: (0, 0)),
            pl.BlockSpec((_BLK, _BLK), lambda i: (0, 0)),
        ],
        out_specs=[
            pl.BlockSpec((_BLK, _DM), lambda i: (i, 0)),
            pl.BlockSpec((_BLK, 1), lambda i: (i, 0)),
            pl.BlockSpec((_BLK, 1), lambda i: (i, 0)),
            pl.BlockSpec((_BLK, 1), lambda i: (i, 0)),
            pl.BlockSpec((_BLK, 1), lambda i: (i, 0)),
            pl.BlockSpec((_BLK, 1), lambda i: (i, 0)),
            pl.BlockSpec((_BLK, 1), lambda i: (i, 0)),
            pl.BlockSpec((1, _E), lambda i: (0, 0)),
            pl.BlockSpec((1, _E), lambda i: (0, 0)),
            pl.BlockSpec((1, 1), lambda i: (0, 0)),
        ],
        out_shape=[
            jax.ShapeDtypeStruct((_T, _DM), jnp.float32),
            jax.ShapeDtypeStruct((_T, 1), jnp.int32),
            jax.ShapeDtypeStruct((_T, 1), jnp.int32),
            jax.ShapeDtypeStruct((_T, 1), jnp.float32),
            jax.ShapeDtypeStruct((_T, 1), jnp.float32),
            jax.ShapeDtypeStruct((_T, 1), jnp.int32),
            jax.ShapeDtypeStruct((_T, 1), jnp.int32),
            jax.ShapeDtypeStruct((1, _E), jnp.float32),
            jax.ShapeDtypeStruct((1, _E), jnp.float32),
            jax.ShapeDtypeStruct((1, 1), jnp.float32),
        ],
        scratch_shapes=[pltpu.VMEM((8, 128), jnp.float32)],
    )(patches, wp, posrep, wg, tril)


def _finish_body(cnt_ref, sp_ref, sl_ref, i0_ref, i1_ref, r0_ref, r1_ref,
                 up_ref, d0_ref, d1_ref, te_ref, aux_ref):
    cnt = cnt_ref[...]
    cap = jnp.floor((cnt + 127.0) * (1.0 / 128.0)) * 128.0
    po = jnp.dot(cap, up_ref[...], precision=lax.Precision.HIGHEST,
                 preferred_element_type=jnp.float32)
    iota8 = lax.broadcasted_iota(jnp.int32, (_T, _E), 1)
    one0 = (iota8 == i0_ref[...]).astype(jnp.float32)
    one1 = (iota8 == i1_ref[...]).astype(jnp.float32)
    d0 = jnp.sum(one0 * po, axis=1, keepdims=True).astype(jnp.int32) + r0_ref[...]
    d1 = jnp.sum(one1 * po, axis=1, keepdims=True).astype(jnp.int32) + r1_ref[...]
    d0_ref[...] = d0
    d1_ref[...] = d1
    ts = lax.broadcasted_iota(jnp.int32, (_NTILE, _E), 0).astype(jnp.float32) * float(_TILE)
    ge = (ts >= po).astype(jnp.int32)
    te_ref[...] = jnp.sum(ge, axis=1, keepdims=True) - 1
    balance = float(_E) * jnp.sum(cnt * sp_ref[...]) * (1.0 / (_T * _T))
    aux_ref[...] = 0.01 * balance + 0.001 * (sl_ref[...] * (1.0 / _T))


def _finish(cnt, sp, sl, i0, i1, r0, r1):
    up = jnp.asarray(np.triu(np.ones((_E, _E), np.float32), 1))
    return pl.pallas_call(
        _finish_body,
        out_shape=[
            jax.ShapeDtypeStruct((_T, 1), jnp.int32),
            jax.ShapeDtypeStruct((_T, 1), jnp.int32),
            jax.ShapeDtypeStruct((_NTILE, 1), jnp.int32),
            jax.ShapeDtypeStruct((1, 1), jnp.float32),
        ],
    )(cnt, sp, sl, i0, i1, r0, r1, up)


def _cast_body(w1_ref, w2_ref, o1_ref, o2_ref):
    o1_ref[...] = w1_ref[...].astype(jnp.bfloat16)
    o2_ref[...] = w2_ref[...].astype(jnp.bfloat16)


def _cast_weights(w1, w2):
    return pl.pallas_call(
        _cast_body,
        grid=(_E,),
        in_specs=[pl.BlockSpec((None, _DM, _HID), lambda i: (i, 0, 0)),
                  pl.BlockSpec((None, _HID, _DM), lambda i: (i, 0, 0))],
        out_specs=[pl.BlockSpec((None, _DM, _HID), lambda i: (i, 0, 0)),
                   pl.BlockSpec((None, _HID, _DM), lambda i: (i, 0, 0))],
        out_shape=[jax.ShapeDtypeStruct((_E, _DM, _HID), jnp.bfloat16),
                   jax.ShapeDtypeStruct((_E, _HID, _DM), jnp.bfloat16)],
    )(w1, w2)


def _sc_dispatch(emb, ga, gb, dest_a, dest_b):
    mesh = plsc.VectorSubcoreMesh(core_axis_name="c", subcore_axis_name="s")

    @functools.partial(
        pl.kernel,
        compiler_params=pltpu.CompilerParams(use_tc_tiling_on_sc=True),
        out_type=[jax.ShapeDtypeStruct((_GCAP, _DM), jnp.float32),
                  jax.ShapeDtypeStruct((_GCAP, 128), jnp.float32)],
        mesh=mesh,
        scratch_types=[
            pltpu.VMEM((_CHUNK, _DM), jnp.float32),
            pltpu.VMEM((_CHUNK, 128), jnp.float32),
            pltpu.VMEM((_CHUNK, 128), jnp.float32),
            pltpu.VMEM((_CHUNK,), jnp.int32),
            pltpu.VMEM((_CHUNK,), jnp.int32),
            pltpu.SemaphoreType.DMA,
        ],
    )
    def k(emb_hbm, ga_hbm, gb_hbm, da_hbm, db_hbm, xg_hbm, xgate_hbm,
          emb_v, ga_v, gb_v, ia_v, ib_v, sem):
        wid = lax.axis_index("s") * 2 + lax.axis_index("c")
        base = wid * _CHUNK
        pltpu.sync_copy(emb_hbm.at[pl.ds(base, _CHUNK)], emb_v)
        pltpu.sync_copy(ga_hbm.at[pl.ds(base, _CHUNK)], ga_v)
        pltpu.sync_copy(gb_hbm.at[pl.ds(base, _CHUNK)], gb_v)
        pltpu.sync_copy(da_hbm.at[wid], ia_v)
        pltpu.sync_copy(db_hbm.at[wid], ib_v)
        c1 = pltpu.async_copy(emb_v, xg_hbm.at[ia_v], sem)
        c2 = pltpu.async_copy(emb_v, xg_hbm.at[ib_v], sem)
        c3 = pltpu.async_copy(ga_v, xgate_hbm.at[ia_v], sem)
        c4 = pltpu.async_copy(gb_v, xgate_hbm.at[ib_v], sem)
        c1.wait()
        c2.wait()
        c3.wait()
        c4.wait()

    return k(emb, ga, gb, dest_a, dest_b)


def _ffn_body(te_ref, x_ref, g_ref, w1_ref, b1_ref, w2_ref, b2_ref, y_ref):
    x = x_ref[...].astype(jnp.bfloat16)
    h = jnp.dot(x, w1_ref[...], preferred_element_type=jnp.float32) + b1_ref[...]
    h = h * 0.5 * (1.0 + lax.erf(h * np.float32(1.0 / np.sqrt(2.0))))
    h = h.astype(jnp.bfloat16)
    y = jnp.dot(h, w2_ref[...], preferred_element_type=jnp.float32) + b2_ref[...]
    y_ref[...] = y * g_ref[:, 0:1]


def _ffn(tile_expert, xg, xgate, w1, b1, w2, b2):
    grid_spec = pltpu.PrefetchScalarGridSpec(
        num_scalar_prefetch=1,
        grid=(_NTILE,),
        in_specs=[
            pl.BlockSpec((_TILE, _DM), lambda i, te: (i, 0)),
            pl.BlockSpec((_TILE, 128), lambda i, te: (i, 0)),
            pl.BlockSpec((None, _DM, _HID), lambda i, te: (te[i], 0, 0)),
            pl.BlockSpec((None, 1, _HID), lambda i, te: (te[i], 0, 0)),
            pl.BlockSpec((None, _HID, _DM), lambda i, te: (te[i], 0, 0)),
            pl.BlockSpec((None, 1, _DM), lambda i, te: (te[i], 0, 0)),
        ],
        out_specs=pl.BlockSpec((_TILE, _DM), lambda i, te: (i, 0)),
    )
    w1b, w2b = _cast_weights(w1, w2)
    return pl.pallas_call(
        _ffn_body,
        grid_spec=grid_spec,
        out_shape=jax.ShapeDtypeStruct((_GCAP, _DM), jnp.float32),
    )(tile_expert, xg, xgate,
      w1b, b1.reshape(_E, 1, _HID),
      w2b, b2.reshape(_E, 1, _DM))


def _sc_combine(y, da_p, db_p):
    mesh = plsc.VectorSubcoreMesh(core_axis_name="c", subcore_axis_name="s")

    @functools.partial(
        pl.kernel,
        compiler_params=pltpu.CompilerParams(use_tc_tiling_on_sc=True),
        out_type=[jax.ShapeDtypeStruct((_T, _DM), jnp.float32),
                  jax.ShapeDtypeStruct((_T, _DM), jnp.float32)],
        mesh=mesh,
        scratch_types=[
            pltpu.VMEM((_CHUNK, _DM), jnp.float32),
            pltpu.VMEM((_CHUNK,), jnp.int32),
            pltpu.VMEM((_CHUNK,), jnp.int32),
            pltpu.SemaphoreType.DMA,
        ],
    )
    def k(y_hbm, da_hbm, db_hbm, ya_hbm, yb_hbm, buf_v, ia_v, ib_v, sem):
        wid = lax.axis_index("s") * 2 + lax.axis_index("c")
        base = wid * _CHUNK
        pltpu.sync_copy(da_hbm.at[wid], ia_v)
        pltpu.sync_copy(db_hbm.at[wid], ib_v)
        pltpu.async_copy(y_hbm.at[ia_v], buf_v, sem).wait()
        pltpu.sync_copy(buf_v, ya_hbm.at[pl.ds(base, _CHUNK)])
        pltpu.async_copy(y_hbm.at[ib_v], buf_v, sem).wait()
        pltpu.sync_copy(buf_v, yb_hbm.at[pl.ds(base, _CHUNK)])

    return k(y, da_p, db_p)


def _head_body(ya_ref, yb_ref, wh_ref, bh_ref, out_ref):
    i = pl.program_id(0)

    @pl.when(i == 0)
    def _():
        out_ref[...] = jnp.broadcast_to(bh_ref[...], (_BV, _PRED))

    s = ya_ref[...] + yb_ref[...]
    acc = out_ref[...]
    for k in range(_HG):
        acc = acc + jnp.dot(s[k * _BV:(k + 1) * _BV, :], wh_ref[k],
                            preferred_element_type=jnp.float32)
    out_ref[...] = acc


def _head(yan, ybn, wh2n, bh2):
    return pl.pallas_call(
        _head_body,
        grid=(_NP // _HG,),
        in_specs=[
            pl.BlockSpec((_HG * _BV, _DM), lambda i: (i, 0)),
            pl.BlockSpec((_HG * _BV, _DM), lambda i: (i, 0)),
            pl.BlockSpec((_HG, _DM, _PRED), lambda i: (i, 0, 0)),
            pl.BlockSpec((1, _PRED), lambda i: (0, 0)),
        ],
        out_specs=pl.BlockSpec((_BV, _PRED), lambda i: (0, 0)),
        out_shape=jax.ShapeDtypeStruct((_BV, _PRED), jnp.float32),
    )(yan, ybn, wh2n, bh2)


def kernel(x_enc, x_mark_enc, x_dec, x_mark_dec, W_patch, Wg, W1, b1, W2, b2, Wh, bh):
    means = jnp.mean(x_enc, axis=1, keepdims=True)
    xe = x_enc - means
    stdev = jnp.sqrt(jnp.var(xe, axis=1, keepdims=True) + 1e-5)
    xe = xe / stdev
    x = jnp.transpose(xe, (0, 2, 1))
    xpad = jnp.concatenate([x, jnp.repeat(x[..., -1:], _ST, axis=-1)], axis=-1)
    xc = xpad.reshape(_BV, _L // _ST + 1, _ST)
    patches = jnp.concatenate([xc[:, :-1, :], xc[:, 1:, :]], axis=-1)
    patches = patches.transpose(1, 0, 2).reshape(_T, _PLEN)

    posrep = jnp.asarray(_POS_NMAJOR)
    tril = jnp.asarray(np.tril(np.ones((_BLK, _BLK), np.float32)))
    (emb, i0, i1, v0, v1, r0, r1, cnt, sp, sl) = _prologue(
        patches, W_patch, posrep, Wg, tril)

    dest0, dest1, te, aux = _finish(cnt, sp, sl, i0, i1, r0, r1)
    tile_expert = te.reshape(_NTILE)

    ga = jnp.broadcast_to(v0, (_T, 128))
    gb = jnp.broadcast_to(v1, (_T, 128))
    xg, xgate = _sc_dispatch(emb, ga, gb,
                             dest0.reshape(_NW, _CHUNK),
                             dest1.reshape(_NW, _CHUNK))

    y = _ffn(tile_expert, xg, xgate, W1, b1, W2, b2)

    yan, ybn = _sc_combine(y,
                           dest0.reshape(_NW, _CHUNK),
                           dest1.reshape(_NW, _CHUNK))

    wh2n = Wh.reshape(_DM, _NP, _PRED).transpose(1, 0, 2)
    dec56 = _head(yan, ybn, wh2n, bh.reshape(1, _PRED))
    dec = dec56.reshape(_B, _NV, _PRED).transpose(0, 2, 1)
    dec = dec * stdev[:, 0, :][:, None, :] + means[:, 0, :][:, None, :]

    return dec, aux[0, 0]


# speedup vs baseline: 3.0514x; 1.0354x over previous
import functools

import numpy as np
import jax
import jax.numpy as jnp
from jax import lax
from jax.experimental import pallas as pl
from jax.experimental.pallas import tpu as pltpu
from jax.experimental.pallas import tpu_sc as plsc

_B, _L, _NV, _DM = 8, 512, 7, 768
_PLEN, _ST, _E = 16, 8, 8
_HID, _PRED, _NP = 1536, 96, 64
_BV = _B * _NV
_T = _BV * _NP
_BLK = 896
_NBLK = _T // _BLK
_TILE = 256
_GCAP = 9216
_NTILE = _GCAP // _TILE
_NW = 32
_CHUNK = _T // _NW
_HG = 8


def _posemb_np():
    position = np.arange(_NP, dtype=np.float32)[:, None]
    div = np.exp(np.arange(0, _DM, 2, dtype=np.float32) * -(np.log(10000.0) / _DM))
    pe = np.zeros((_NP, _DM), np.float32)
    pe[:, 0::2] = np.sin(position * div)
    pe[:, 1::2] = np.cos(position * div)
    return pe


_POS_NMAJOR = np.repeat(_posemb_np(), _BV, axis=0)


def _prologue_body(patches_ref, wp_ref, pos_ref, wg_ref, tril_ref,
                   emb_ref, i0_ref, i1_ref, v0_ref, v1_ref, r0_ref, r1_ref,
                   cnt_ref, sp_ref, sl_ref, acc):
    i = pl.program_id(0)

    @pl.when(i == 0)
    def _():
        acc[...] = jnp.zeros((8, 128), jnp.float32)

    p = patches_ref[...]
    emb = jnp.dot(p, wp_ref[...], preferred_element_type=jnp.float32) + pos_ref[...]
    emb_ref[...] = emb
    logits = jnp.dot(emb, wg_ref[...], preferred_element_type=jnp.float32)
    m = jnp.max(logits, axis=1, keepdims=True)
    ex = jnp.exp(logits - m)
    s = jnp.sum(ex, axis=1, keepdims=True)
    probs = ex / s
    lse = jnp.log(s) + m
    iota8 = lax.broadcasted_iota(jnp.int32, (_BLK, _E), 1)
    v0 = jnp.max(probs, axis=1, keepdims=True)
    i0 = jnp.min(jnp.where(probs == v0, iota8, _E), axis=1, keepdims=True)
    m0 = iota8 == i0
    pm = jnp.where(m0, -jnp.inf, probs)
    v1 = jnp.max(pm, axis=1, keepdims=True)
    i1 = jnp.min(jnp.where(pm == v1, iota8, _E), axis=1, keepdims=True)
    m1 = iota8 == i1
    d = m0.astype(jnp.float32) + m1.astype(jnp.float32)
    incl = jnp.dot(tril_ref[...], d, precision=lax.Precision.HIGHEST,
                   preferred_element_type=jnp.float32)
    cnt = acc[0:1, 0:_E]
    excl = incl - d + cnt
    r0 = jnp.sum(excl * m0.astype(jnp.float32), axis=1, keepdims=True)
    r1 = jnp.sum(excl * m1.astype(jnp.float32), axis=1, keepdims=True)
    i0_ref[...] = i0
    i1_ref[...] = i1
    v0_ref[...] = v0
    v1_ref[...] = v1
    r0_ref[...] = r0.astype(jnp.int32)
    r1_ref[...] = r1.astype(jnp.int32)
    ncnt = cnt + jnp.sum(d, axis=0, keepdims=True)
    nsp = acc[1:2, 0:_E] + jnp.sum(probs, axis=0, keepdims=True)
    nsl = acc[2:3, 0:1] + jnp.sum(lse * lse)
    acc[0:1, 0:_E] = ncnt
    acc[1:2, 0:_E] = nsp
    acc[2:3, 0:1] = nsl
    cnt_ref[...] = ncnt
    sp_ref[...] = nsp
    sl_ref[...] = nsl


def _prologue(patches, wp, posrep, wg, tril):
    return pl.pallas_call(
        _prologue_body,
        grid=(_NBLK,),
        in_specs=[
            pl.BlockSpec((_BLK, _PLEN), lambda i: (i, 0)),
            pl.BlockSpec((_PLEN, _DM), lambda i: (0, 0)),
            pl.BlockSpec((_BLK, _DM), lambda i: (i, 0)),
            pl.BlockSpec((_DM, _E), lambda i: (0, 0)),
            pl.BlockSpec((_BLK, _BLK), lambda i: (0, 0)),
        ],
        out_specs=[
            pl.BlockSpec((_BLK, _DM), lambda i: (i, 0)),
            pl.BlockSpec((_BLK, 1), lambda i: (i, 0)),
            pl.BlockSpec((_BLK, 1), lambda i: (i, 0)),
            pl.BlockSpec((_BLK, 1), lambda i: (i, 0)),
            pl.BlockSpec((_BLK, 1), lambda i: (i, 0)),
            pl.BlockSpec((_BLK, 1), lambda i: (i, 0)),
            pl.BlockSpec((_BLK, 1), lambda i: (i, 0)),
            pl.BlockSpec((1, _E), lambda i: (0, 0)),
            pl.BlockSpec((1, _E), lambda i: (0, 0)),
            pl.BlockSpec((1, 1), lambda i: (0, 0)),
        ],
        out_shape=[
            jax.ShapeDtypeStruct((_T, _DM), jnp.float32),
            jax.ShapeDtypeStruct((_T, 1), jnp.int32),
            jax.ShapeDtypeStruct((_T, 1), jnp.int32),
            jax.ShapeDtypeStruct((_T, 1), jnp.float32),
            jax.ShapeDtypeStruct((_T, 1), jnp.float32),
            jax.ShapeDtypeStruct((_T, 1), jnp.int32),
            jax.ShapeDtypeStruct((_T, 1), jnp.int32),
            jax.ShapeDtypeStruct((1, _E), jnp.float32),
            jax.ShapeDtypeStruct((1, _E), jnp.float32),
            jax.ShapeDtypeStruct((1, 1), jnp.float32),
        ],
        scratch_shapes=[pltpu.VMEM((8, 128), jnp.float32)],
    )(patches, wp, posrep, wg, tril)


def _finish_body(cnt_ref, sp_ref, sl_ref, i0_ref, i1_ref, r0_ref, r1_ref,
                 up_ref, d0_ref, d1_ref, te_ref, aux_ref):
    cnt = cnt_ref[...]
    cap = jnp.floor((cnt + float(_TILE - 1)) * (1.0 / _TILE)) * float(_TILE)
    po = jnp.dot(cap, up_ref[...], precision=lax.Precision.HIGHEST,
                 preferred_element_type=jnp.float32)
    iota8 = lax.broadcasted_iota(jnp.int32, (_T, _E), 1)
    one0 = (iota8 == i0_ref[...]).astype(jnp.float32)
    one1 = (iota8 == i1_ref[...]).astype(jnp.float32)
    d0 = jnp.sum(one0 * po, axis=1, keepdims=True).astype(jnp.int32) + r0_ref[...]
    d1 = jnp.sum(one1 * po, axis=1, keepdims=True).astype(jnp.int32) + r1_ref[...]
    d0_ref[...] = d0
    d1_ref[...] = d1
    ts = lax.broadcasted_iota(jnp.int32, (_NTILE, _E), 0).astype(jnp.float32) * float(_TILE)
    ge = (ts >= po).astype(jnp.int32)
    te_ref[...] = jnp.sum(ge, axis=1, keepdims=True) - 1
    balance = float(_E) * jnp.sum(cnt * sp_ref[...]) * (1.0 / (_T * _T))
    aux_ref[...] = 0.01 * balance + 0.001 * (sl_ref[...] * (1.0 / _T))


def _finish(cnt, sp, sl, i0, i1, r0, r1):
    up = jnp.asarray(np.triu(np.ones((_E, _E), np.float32), 1))
    return pl.pallas_call(
        _finish_body,
        out_shape=[
            jax.ShapeDtypeStruct((_T, 1), jnp.int32),
            jax.ShapeDtypeStruct((_T, 1), jnp.int32),
            jax.ShapeDtypeStruct((_NTILE, 1), jnp.int32),
            jax.ShapeDtypeStruct((1, 1), jnp.float32),
        ],
    )(cnt, sp, sl, i0, i1, r0, r1, up)


def _cast_body(w1_ref, w2_ref, o1_ref, o2_ref):
    o1_ref[...] = w1_ref[...].astype(jnp.bfloat16)
    o2_ref[...] = w2_ref[...].astype(jnp.bfloat16)


def _cast_weights(w1, w2):
    return pl.pallas_call(
        _cast_body,
        grid=(_E,),
        in_specs=[pl.BlockSpec((None, _DM, _HID), lambda i: (i, 0, 0)),
                  pl.BlockSpec((None, _HID, _DM), lambda i: (i, 0, 0))],
        out_specs=[pl.BlockSpec((None, _DM, _HID), lambda i: (i, 0, 0)),
                   pl.BlockSpec((None, _HID, _DM), lambda i: (i, 0, 0))],
        out_shape=[jax.ShapeDtypeStruct((_E, _DM, _HID), jnp.bfloat16),
                   jax.ShapeDtypeStruct((_E, _HID, _DM), jnp.bfloat16)],
    )(w1, w2)


def _sc_dispatch(emb, ga, gb, dest_a, dest_b):
    mesh = plsc.VectorSubcoreMesh(core_axis_name="c", subcore_axis_name="s")

    @functools.partial(
        pl.kernel,
        compiler_params=pltpu.CompilerParams(use_tc_tiling_on_sc=True),
        out_type=[jax.ShapeDtypeStruct((_GCAP, _DM), jnp.float32),
                  jax.ShapeDtypeStruct((_GCAP, 128), jnp.float32)],
        mesh=mesh,
        scratch_types=[
            pltpu.VMEM((_CHUNK, _DM), jnp.float32),
            pltpu.VMEM((_CHUNK, 128), jnp.float32),
            pltpu.VMEM((_CHUNK, 128), jnp.float32),
            pltpu.VMEM((_CHUNK,), jnp.int32),
            pltpu.VMEM((_CHUNK,), jnp.int32),
            pltpu.SemaphoreType.DMA,
        ],
    )
    def k(emb_hbm, ga_hbm, gb_hbm, da_hbm, db_hbm, xg_hbm, xgate_hbm,
          emb_v, ga_v, gb_v, ia_v, ib_v, sem):
        wid = lax.axis_index("s") * 2 + lax.axis_index("c")
        base = wid * _CHUNK
        pltpu.sync_copy(emb_hbm.at[pl.ds(base, _CHUNK)], emb_v)
        pltpu.sync_copy(ga_hbm.at[pl.ds(base, _CHUNK)], ga_v)
        pltpu.sync_copy(gb_hbm.at[pl.ds(base, _CHUNK)], gb_v)
        pltpu.sync_copy(da_hbm.at[wid], ia_v)
        pltpu.sync_copy(db_hbm.at[wid], ib_v)
        c1 = pltpu.async_copy(emb_v, xg_hbm.at[ia_v], sem)
        c2 = pltpu.async_copy(emb_v, xg_hbm.at[ib_v], sem)
        c3 = pltpu.async_copy(ga_v, xgate_hbm.at[ia_v], sem)
        c4 = pltpu.async_copy(gb_v, xgate_hbm.at[ib_v], sem)
        c1.wait()
        c2.wait()
        c3.wait()
        c4.wait()

    return k(emb, ga, gb, dest_a, dest_b)


def _ffn_body(te_ref, x_ref, g_ref, w1_ref, b1_ref, w2_ref, b2_ref, y_ref):
    x = x_ref[...].astype(jnp.bfloat16)
    h = jnp.dot(x, w1_ref[...], preferred_element_type=jnp.float32) + b1_ref[...]
    h = h * 0.5 * (1.0 + lax.erf(h * np.float32(1.0 / np.sqrt(2.0))))
    h = h.astype(jnp.bfloat16)
    y = jnp.dot(h, w2_ref[...], preferred_element_type=jnp.float32) + b2_ref[...]
    y_ref[...] = y * g_ref[:, 0:1]


def _ffn(tile_expert, xg, xgate, w1, b1, w2, b2):
    grid_spec = pltpu.PrefetchScalarGridSpec(
        num_scalar_prefetch=1,
        grid=(_NTILE,),
        in_specs=[
            pl.BlockSpec((_TILE, _DM), lambda i, te: (i, 0)),
            pl.BlockSpec((_TILE, 128), lambda i, te: (i, 0)),
            pl.BlockSpec((None, _DM, _HID), lambda i, te: (te[i], 0, 0)),
            pl.BlockSpec((None, 1, _HID), lambda i, te: (te[i], 0, 0)),
            pl.BlockSpec((None, _HID, _DM), lambda i, te: (te[i], 0, 0)),
            pl.BlockSpec((None, 1, _DM), lambda i, te: (te[i], 0, 0)),
        ],
        out_specs=pl.BlockSpec((_TILE, _DM), lambda i, te: (i, 0)),
    )
    w1b, w2b = _cast_weights(w1, w2)
    return pl.pallas_call(
        _ffn_body,
        grid_spec=grid_spec,
        out_shape=jax.ShapeDtypeStruct((_GCAP, _DM), jnp.float32),
    )(tile_expert, xg, xgate,
      w1b, b1.reshape(_E, 1, _HID),
      w2b, b2.reshape(_E, 1, _DM))


def _sc_combine(y, da_p, db_p):
    mesh = plsc.VectorSubcoreMesh(core_axis_name="c", subcore_axis_name="s")

    @functools.partial(
        pl.kernel,
        compiler_params=pltpu.CompilerParams(use_tc_tiling_on_sc=True),
        out_type=[jax.ShapeDtypeStruct((_T, _DM), jnp.float32),
                  jax.ShapeDtypeStruct((_T, _DM), jnp.float32)],
        mesh=mesh,
        scratch_types=[
            pltpu.VMEM((_CHUNK, _DM), jnp.float32),
            pltpu.VMEM((_CHUNK,), jnp.int32),
            pltpu.VMEM((_CHUNK,), jnp.int32),
            pltpu.SemaphoreType.DMA,
        ],
    )
    def k(y_hbm, da_hbm, db_hbm, ya_hbm, yb_hbm, buf_v, ia_v, ib_v, sem):
        wid = lax.axis_index("s") * 2 + lax.axis_index("c")
        base = wid * _CHUNK
        pltpu.sync_copy(da_hbm.at[wid], ia_v)
        pltpu.sync_copy(db_hbm.at[wid], ib_v)
        pltpu.async_copy(y_hbm.at[ia_v], buf_v, sem).wait()
        pltpu.sync_copy(buf_v, ya_hbm.at[pl.ds(base, _CHUNK)])
        pltpu.async_copy(y_hbm.at[ib_v], buf_v, sem).wait()
        pltpu.sync_copy(buf_v, yb_hbm.at[pl.ds(base, _CHUNK)])

    return k(y, da_p, db_p)


def _head_body(ya_ref, yb_ref, wh_ref, bh_ref, out_ref):
    i = pl.program_id(0)

    @pl.when(i == 0)
    def _():
        out_ref[...] = jnp.broadcast_to(bh_ref[...], (_BV, _PRED))

    s = ya_ref[...] + yb_ref[...]
    acc = out_ref[...]
    for k in range(_HG):
        acc = acc + jnp.dot(s[k * _BV:(k + 1) * _BV, :], wh_ref[k],
                            preferred_element_type=jnp.float32)
    out_ref[...] = acc


def _head(yan, ybn, wh2n, bh2):
    return pl.pallas_call(
        _head_body,
        grid=(_NP // _HG,),
        in_specs=[
            pl.BlockSpec((_HG * _BV, _DM), lambda i: (i, 0)),
            pl.BlockSpec((_HG * _BV, _DM), lambda i: (i, 0)),
            pl.BlockSpec((_HG, _DM, _PRED), lambda i: (i, 0, 0)),
            pl.BlockSpec((1, _PRED), lambda i: (0, 0)),
        ],
        out_specs=pl.BlockSpec((_BV, _PRED), lambda i: (0, 0)),
        out_shape=jax.ShapeDtypeStruct((_BV, _PRED), jnp.float32),
    )(yan, ybn, wh2n, bh2)


def kernel(x_enc, x_mark_enc, x_dec, x_mark_dec, W_patch, Wg, W1, b1, W2, b2, Wh, bh):
    means = jnp.mean(x_enc, axis=1, keepdims=True)
    xe = x_enc - means
    stdev = jnp.sqrt(jnp.var(xe, axis=1, keepdims=True) + 1e-5)
    xe = xe / stdev
    x = jnp.transpose(xe, (0, 2, 1))
    xpad = jnp.concatenate([x, jnp.repeat(x[..., -1:], _ST, axis=-1)], axis=-1)
    xc = xpad.reshape(_BV, _L // _ST + 1, _ST)
    patches = jnp.concatenate([xc[:, :-1, :], xc[:, 1:, :]], axis=-1)
    patches = patches.transpose(1, 0, 2).reshape(_T, _PLEN)

    posrep = jnp.asarray(_POS_NMAJOR)
    tril = jnp.asarray(np.tril(np.ones((_BLK, _BLK), np.float32)))
    (emb, i0, i1, v0, v1, r0, r1, cnt, sp, sl) = _prologue(
        patches, W_patch, posrep, Wg, tril)

    dest0, dest1, te, aux = _finish(cnt, sp, sl, i0, i1, r0, r1)
    tile_expert = te.reshape(_NTILE)

    ga = jnp.broadcast_to(v0, (_T, 128))
    gb = jnp.broadcast_to(v1, (_T, 128))
    xg, xgate = _sc_dispatch(emb, ga, gb,
                             dest0.reshape(_NW, _CHUNK),
                             dest1.reshape(_NW, _CHUNK))

    y = _ffn(tile_expert, xg, xgate, W1, b1, W2, b2)

    yan, ybn = _sc_combine(y,
                           dest0.reshape(_NW, _CHUNK),
                           dest1.reshape(_NW, _CHUNK))

    wh2n = Wh.reshape(_DM, _NP, _PRED).transpose(1, 0, 2)
    dec56 = _head(yan, ybn, wh2n, bh.reshape(1, _PRED))
    dec = dec56.reshape(_B, _NV, _PRED).transpose(0, 2, 1)
    dec = dec * stdev[:, 0, :][:, None, :] + means[:, 0, :][:, None, :]

    return dec, aux[0, 0]


# speedup vs baseline: 3.1697x; 1.0387x over previous
import functools

import numpy as np
import jax
import jax.numpy as jnp
from jax import lax
from jax.experimental import pallas as pl
from jax.experimental.pallas import tpu as pltpu
from jax.experimental.pallas import tpu_sc as plsc

_B, _L, _NV, _DM = 8, 512, 7, 768
_PLEN, _ST, _E = 16, 8, 8
_HID, _PRED, _NP = 1536, 96, 64
_BV = _B * _NV
_T = _BV * _NP
_BLK = 512
_NBLK = _T // _BLK
_TILE = 256
_GCAP = 9216
_NTILE = _GCAP // _TILE
_NW = 32
_CHUNK = _T // _NW
_HG = 8


def _posemb_np():
    position = np.arange(_NP, dtype=np.float32)[:, None]
    div = np.exp(np.arange(0, _DM, 2, dtype=np.float32) * -(np.log(10000.0) / _DM))
    pe = np.zeros((_NP, _DM), np.float32)
    pe[:, 0::2] = np.sin(position * div)
    pe[:, 1::2] = np.cos(position * div)
    return pe


_POS_NMAJOR = np.repeat(_posemb_np(), _BV, axis=0)


def _prologue_body(patches_ref, wp_ref, pos_ref, wg_ref, tril_ref,
                   emb_ref, i0_ref, i1_ref, v0_ref, v1_ref, r0_ref, r1_ref,
                   cnt_ref, sp_ref, sl_ref, acc):
    i = pl.program_id(0)

    @pl.when(i == 0)
    def _():
        acc[...] = jnp.zeros((8, 128), jnp.float32)

    p = patches_ref[...]
    emb = jnp.dot(p, wp_ref[...], preferred_element_type=jnp.float32) + pos_ref[...]
    emb_ref[...] = emb
    logits = jnp.dot(emb, wg_ref[...], preferred_element_type=jnp.float32)
    m = jnp.max(logits, axis=1, keepdims=True)
    ex = jnp.exp(logits - m)
    s = jnp.sum(ex, axis=1, keepdims=True)
    probs = ex / s
    lse = jnp.log(s) + m
    iota8 = lax.broadcasted_iota(jnp.int32, (_BLK, _E), 1)
    v0 = jnp.max(probs, axis=1, keepdims=True)
    i0 = jnp.min(jnp.where(probs == v0, iota8, _E), axis=1, keepdims=True)
    m0 = iota8 == i0
    pm = jnp.where(m0, -jnp.inf, probs)
    v1 = jnp.max(pm, axis=1, keepdims=True)
    i1 = jnp.min(jnp.where(pm == v1, iota8, _E), axis=1, keepdims=True)
    m1 = iota8 == i1
    d = m0.astype(jnp.float32) + m1.astype(jnp.float32)
    incl = jnp.dot(tril_ref[...], d, precision=lax.Precision.HIGHEST,
                   preferred_element_type=jnp.float32)
    cnt = acc[0:1, 0:_E]
    excl = incl - d + cnt
    r0 = jnp.sum(excl * m0.astype(jnp.float32), axis=1, keepdims=True)
    r1 = jnp.sum(excl * m1.astype(jnp.float32), axis=1, keepdims=True)
    i0_ref[...] = i0
    i1_ref[...] = i1
    v0_ref[...] = v0
    v1_ref[...] = v1
    r0_ref[...] = r0.astype(jnp.int32)
    r1_ref[...] = r1.astype(jnp.int32)
    ncnt = cnt + jnp.sum(d, axis=0, keepdims=True)
    nsp = acc[1:2, 0:_E] + jnp.sum(probs, axis=0, keepdims=True)
    nsl = acc[2:3, 0:1] + jnp.sum(lse * lse)
    acc[0:1, 0:_E] = ncnt
    acc[1:2, 0:_E] = nsp
    acc[2:3, 0:1] = nsl
    cnt_ref[...] = ncnt
    sp_ref[...] = nsp
    sl_ref[...] = nsl


def _prologue(patches, wp, posrep, wg, tril):
    return pl.pallas_call(
        _prologue_body,
        grid=(_NBLK,),
        in_specs=[
            pl.BlockSpec((_BLK, _PLEN), lambda i: (i, 0)),
            pl.BlockSpec((_PLEN, _DM), lambda i: (0, 0)),
            pl.BlockSpec((_BLK, _DM), lambda i: (i, 0)),
            pl.BlockSpec((_DM, _E), lambda i: (0, 0)),
            pl.BlockSpec((_BLK, _BLK), lambda i: (0, 0)),
        ],
        out_specs=[
            pl.BlockSpec((_BLK, _DM), lambda i: (i, 0)),
            pl.BlockSpec((_BLK, 1), lambda i: (i, 0)),
            pl.BlockSpec((_BLK, 1), lambda i: (i, 0)),
            pl.BlockSpec((_BLK, 1), lambda i: (i, 0)),
            pl.BlockSpec((_BLK, 1), lambda i: (i, 0)),
            pl.BlockSpec((_BLK, 1), lambda i: (i, 0)),
            pl.BlockSpec((_BLK, 1), lambda i: (i, 0)),
            pl.BlockSpec((1, _E), lambda i: (0, 0)),
            pl.BlockSpec((1, _E), lambda i: (0, 0)),
            pl.BlockSpec((1, 1), lambda i: (0, 0)),
        ],
        out_shape=[
            jax.ShapeDtypeStruct((_T, _DM), jnp.float32),
            jax.ShapeDtypeStruct((_T, 1), jnp.int32),
            jax.ShapeDtypeStruct((_T, 1), jnp.int32),
            jax.ShapeDtypeStruct((_T, 1), jnp.float32),
            jax.ShapeDtypeStruct((_T, 1), jnp.float32),
            jax.ShapeDtypeStruct((_T, 1), jnp.int32),
            jax.ShapeDtypeStruct((_T, 1), jnp.int32),
            jax.ShapeDtypeStruct((1, _E), jnp.float32),
            jax.ShapeDtypeStruct((1, _E), jnp.float32),
            jax.ShapeDtypeStruct((1, 1), jnp.float32),
        ],
        scratch_shapes=[pltpu.VMEM((8, 128), jnp.float32)],
    )(patches, wp, posrep, wg, tril)


def _finish_body(cnt_ref, sp_ref, sl_ref, i0_ref, i1_ref, r0_ref, r1_ref,
                 v0_ref, v1_ref,
                 up_ref, d0_ref, d1_ref, te_ref, aux_ref, ga_ref, gb_ref):
    cnt = cnt_ref[...]
    cap = jnp.floor((cnt + float(_TILE - 1)) * (1.0 / _TILE)) * float(_TILE)
    po = jnp.dot(cap, up_ref[...], precision=lax.Precision.HIGHEST,
                 preferred_element_type=jnp.float32)
    iota8 = lax.broadcasted_iota(jnp.int32, (_T, _E), 1)
    one0 = (iota8 == i0_ref[...]).astype(jnp.float32)
    one1 = (iota8 == i1_ref[...]).astype(jnp.float32)
    d0 = jnp.sum(one0 * po, axis=1, keepdims=True).astype(jnp.int32) + r0_ref[...]
    d1 = jnp.sum(one1 * po, axis=1, keepdims=True).astype(jnp.int32) + r1_ref[...]
    d0_ref[...] = d0
    d1_ref[...] = d1
    ts = lax.broadcasted_iota(jnp.int32, (_NTILE, _E), 0).astype(jnp.float32) * float(_TILE)
    ge = (ts >= po).astype(jnp.int32)
    te_ref[...] = jnp.sum(ge, axis=1, keepdims=True) - 1
    balance = float(_E) * jnp.sum(cnt * sp_ref[...]) * (1.0 / (_T * _T))
    aux_ref[...] = 0.01 * balance + 0.001 * (sl_ref[...] * (1.0 / _T))
    ga_ref[...] = jnp.broadcast_to(v0_ref[...], (_T, 128))
    gb_ref[...] = jnp.broadcast_to(v1_ref[...], (_T, 128))


def _finish(cnt, sp, sl, i0, i1, r0, r1, v0, v1):
    up = jnp.asarray(np.triu(np.ones((_E, _E), np.float32), 1))
    return pl.pallas_call(
        _finish_body,
        out_shape=[
            jax.ShapeDtypeStruct((_T, 1), jnp.int32),
            jax.ShapeDtypeStruct((_T, 1), jnp.int32),
            jax.ShapeDtypeStruct((_NTILE, 1), jnp.int32),
            jax.ShapeDtypeStruct((1, 1), jnp.float32),
            jax.ShapeDtypeStruct((_T, 128), jnp.float32),
            jax.ShapeDtypeStruct((_T, 128), jnp.float32),
        ],
    )(cnt, sp, sl, i0, i1, r0, r1, v0, v1, up)


def _cast_body(w1_ref, w2_ref, o1_ref, o2_ref):
    o1_ref[...] = w1_ref[...].astype(jnp.bfloat16)
    o2_ref[...] = w2_ref[...].astype(jnp.bfloat16)


def _cast_weights(w1, w2):
    return pl.pallas_call(
        _cast_body,
        grid=(_E,),
        in_specs=[pl.BlockSpec((None, _DM, _HID), lambda i: (i, 0, 0)),
                  pl.BlockSpec((None, _HID, _DM), lambda i: (i, 0, 0))],
        out_specs=[pl.BlockSpec((None, _DM, _HID), lambda i: (i, 0, 0)),
                   pl.BlockSpec((None, _HID, _DM), lambda i: (i, 0, 0))],
        out_shape=[jax.ShapeDtypeStruct((_E, _DM, _HID), jnp.bfloat16),
                   jax.ShapeDtypeStruct((_E, _HID, _DM), jnp.bfloat16)],
    )(w1, w2)


def _sc_dispatch(emb, ga, gb, dest_a, dest_b):
    mesh = plsc.VectorSubcoreMesh(core_axis_name="c", subcore_axis_name="s")

    @functools.partial(
        pl.kernel,
        out_type=[jax.ShapeDtypeStruct((_GCAP, _DM), jnp.float32),
                  jax.ShapeDtypeStruct((_GCAP, 128), jnp.float32)],
        mesh=mesh,
        scratch_types=[
            pltpu.VMEM((_CHUNK, _DM), jnp.float32),
            pltpu.VMEM((_CHUNK, 128), jnp.float32),
            pltpu.VMEM((_CHUNK, 128), jnp.float32),
            pltpu.VMEM((_CHUNK,), jnp.int32),
            pltpu.VMEM((_CHUNK,), jnp.int32),
            pltpu.SemaphoreType.DMA,
        ],
    )
    def k(emb_hbm, ga_hbm, gb_hbm, da_hbm, db_hbm, xg_hbm, xgate_hbm,
          emb_v, ga_v, gb_v, ia_v, ib_v, sem):
        wid = lax.axis_index("s") * 2 + lax.axis_index("c")
        base = wid * _CHUNK
        pltpu.sync_copy(emb_hbm.at[pl.ds(base, _CHUNK)], emb_v)
        pltpu.sync_copy(ga_hbm.at[pl.ds(base, _CHUNK)], ga_v)
        pltpu.sync_copy(gb_hbm.at[pl.ds(base, _CHUNK)], gb_v)
        pltpu.sync_copy(da_hbm.at[wid], ia_v)
        pltpu.sync_copy(db_hbm.at[wid], ib_v)
        c1 = pltpu.async_copy(emb_v, xg_hbm.at[ia_v], sem)
        c2 = pltpu.async_copy(emb_v, xg_hbm.at[ib_v], sem)
        c3 = pltpu.async_copy(ga_v, xgate_hbm.at[ia_v], sem)
        c4 = pltpu.async_copy(gb_v, xgate_hbm.at[ib_v], sem)
        c1.wait()
        c2.wait()
        c3.wait()
        c4.wait()

    return k(emb, ga, gb, dest_a, dest_b)


def _ffn_body(te_ref, x_ref, g_ref, w1_ref, b1_ref, w2_ref, b2_ref, y_ref):
    x = x_ref[...].astype(jnp.bfloat16)
    h = jnp.dot(x, w1_ref[...], preferred_element_type=jnp.float32) + b1_ref[...]
    h = h * 0.5 * (1.0 + lax.erf(h * np.float32(1.0 / np.sqrt(2.0))))
    h = h.astype(jnp.bfloat16)
    y = jnp.dot(h, w2_ref[...], preferred_element_type=jnp.float32) + b2_ref[...]
    y_ref[...] = y * g_ref[:, 0:1]


def _ffn(tile_expert, xg, xgate, w1, b1, w2, b2):
    grid_spec = pltpu.PrefetchScalarGridSpec(
        num_scalar_prefetch=1,
        grid=(_NTILE,),
        in_specs=[
            pl.BlockSpec((_TILE, _DM), lambda i, te: (i, 0)),
            pl.BlockSpec((_TILE, 128), lambda i, te: (i, 0)),
            pl.BlockSpec((None, _DM, _HID), lambda i, te: (te[i], 0, 0)),
            pl.BlockSpec((None, 1, _HID), lambda i, te: (te[i], 0, 0)),
            pl.BlockSpec((None, _HID, _DM), lambda i, te: (te[i], 0, 0)),
            pl.BlockSpec((None, 1, _DM), lambda i, te: (te[i], 0, 0)),
        ],
        out_specs=pl.BlockSpec((_TILE, _DM), lambda i, te: (i, 0)),
    )
    w1b, w2b = _cast_weights(w1, w2)
    return pl.pallas_call(
        _ffn_body,
        grid_spec=grid_spec,
        out_shape=jax.ShapeDtypeStruct((_GCAP, _DM), jnp.float32),
    )(tile_expert, xg, xgate,
      w1b, b1.reshape(_E, 1, _HID),
      w2b, b2.reshape(_E, 1, _DM))


def _sc_combine(y, da_p, db_p):
    mesh = plsc.VectorSubcoreMesh(core_axis_name="c", subcore_axis_name="s")

    @functools.partial(
        pl.kernel,
        compiler_params=pltpu.CompilerParams(use_tc_tiling_on_sc=True),
        out_type=[jax.ShapeDtypeStruct((_T, _DM), jnp.float32),
                  jax.ShapeDtypeStruct((_T, _DM), jnp.float32)],
        mesh=mesh,
        scratch_types=[
            pltpu.VMEM((_CHUNK, _DM), jnp.float32),
            pltpu.VMEM((_CHUNK,), jnp.int32),
            pltpu.VMEM((_CHUNK,), jnp.int32),
            pltpu.SemaphoreType.DMA,
        ],
    )
    def k(y_hbm, da_hbm, db_hbm, ya_hbm, yb_hbm, buf_v, ia_v, ib_v, sem):
        wid = lax.axis_index("s") * 2 + lax.axis_index("c")
        base = wid * _CHUNK
        pltpu.sync_copy(da_hbm.at[wid], ia_v)
        pltpu.sync_copy(db_hbm.at[wid], ib_v)
        pltpu.async_copy(y_hbm.at[ia_v], buf_v, sem).wait()
        pltpu.sync_copy(buf_v, ya_hbm.at[pl.ds(base, _CHUNK)])
        pltpu.async_copy(y_hbm.at[ib_v], buf_v, sem).wait()
        pltpu.sync_copy(buf_v, yb_hbm.at[pl.ds(base, _CHUNK)])

    return k(y, da_p, db_p)


def _head_body(ya_ref, yb_ref, wh_ref, bh_ref, out_ref):
    i = pl.program_id(0)

    @pl.when(i == 0)
    def _():
        out_ref[...] = jnp.broadcast_to(bh_ref[...], (_BV, _PRED))

    s = ya_ref[...] + yb_ref[...]
    acc = out_ref[...]
    for k in range(_HG):
        acc = acc + jnp.dot(s[k * _BV:(k + 1) * _BV, :], wh_ref[k],
                            preferred_element_type=jnp.float32)
    out_ref[...] = acc


def _head(yan, ybn, wh2n, bh2):
    return pl.pallas_call(
        _head_body,
        grid=(_NP // _HG,),
        in_specs=[
            pl.BlockSpec((_HG * _BV, _DM), lambda i: (i, 0)),
            pl.BlockSpec((_HG * _BV, _DM), lambda i: (i, 0)),
            pl.BlockSpec((_HG, _DM, _PRED), lambda i: (i, 0, 0)),
            pl.BlockSpec((1, _PRED), lambda i: (0, 0)),
        ],
        out_specs=pl.BlockSpec((_BV, _PRED), lambda i: (0, 0)),
        out_shape=jax.ShapeDtypeStruct((_BV, _PRED), jnp.float32),
    )(yan, ybn, wh2n, bh2)


def kernel(x_enc, x_mark_enc, x_dec, x_mark_dec, W_patch, Wg, W1, b1, W2, b2, Wh, bh):
    means = jnp.mean(x_enc, axis=1, keepdims=True)
    xe = x_enc - means
    stdev = jnp.sqrt(jnp.var(xe, axis=1, keepdims=True) + 1e-5)
    xe = xe / stdev
    x = jnp.transpose(xe, (0, 2, 1))
    xpad = jnp.concatenate([x, jnp.repeat(x[..., -1:], _ST, axis=-1)], axis=-1)
    xc = xpad.reshape(_BV, _L // _ST + 1, _ST)
    patches = jnp.concatenate([xc[:, :-1, :], xc[:, 1:, :]], axis=-1)
    patches = patches.transpose(1, 0, 2).reshape(_T, _PLEN)

    posrep = jnp.asarray(_POS_NMAJOR)
    tril = jnp.asarray(np.tril(np.ones((_BLK, _BLK), np.float32)))
    (emb, i0, i1, v0, v1, r0, r1, cnt, sp, sl) = _prologue(
        patches, W_patch, posrep, Wg, tril)

    dest0, dest1, te, aux, ga, gb = _finish(cnt, sp, sl, i0, i1, r0, r1, v0, v1)
    tile_expert = te.reshape(_NTILE)

    xg, xgate = _sc_dispatch(emb, ga, gb,
                             dest0.reshape(_NW, _CHUNK),
                             dest1.reshape(_NW, _CHUNK))

    y = _ffn(tile_expert, xg, xgate, W1, b1, W2, b2)

    yan, ybn = _sc_combine(y,
                           dest0.reshape(_NW, _CHUNK),
                           dest1.reshape(_NW, _CHUNK))

    wh2n = Wh.reshape(_DM, _NP, _PRED).transpose(1, 0, 2)
    dec56 = _head(yan, ybn, wh2n, bh.reshape(1, _PRED))
    dec = dec56.reshape(_B, _NV, _PRED).transpose(0, 2, 1)
    dec = dec * stdev[:, 0, :][:, None, :] + means[:, 0, :][:, None, :]

    return dec, aux[0, 0]
